# baseline (speedup 1.0000x reference)
import jax
import jax.numpy as jnp
from jax import lax
from jax.experimental import pallas as pl
from jax.experimental.pallas import tpu as pltpu
from jax.experimental.pallas import tpu_sc as plsc

N_NODES = 10000
N_EDGES = 160000
D_IN = 256
D_OUT = 256

NC = 2
NS = 16
L = 16

DH = D_OUT // 2
ROWS_PER_TILE = N_NODES // NS
CHUNK = 128
CHUNKS_PER_TILE = 84
EDGES_PAD = NS * CHUNKS_PER_TILE * CHUNK
N_CHUNKS = EDGES_PAD // CHUNK


def _mm_body(h_ref, w_ref, o_ref):
    o_ref[0] = jnp.dot(h_ref[...], w_ref[...],
                       preferred_element_type=jnp.float32)


def _matmul_halves(H, W):
    RB = 400
    grid = (NC, N_NODES // RB)
    return pl.pallas_call(
        _mm_body,
        grid=grid,
        in_specs=[
            pl.BlockSpec((RB, D_IN), lambda c, r: (r, 0)),
            pl.BlockSpec((D_IN, DH), lambda c, r: (0, c)),
        ],
        out_specs=pl.BlockSpec((1, RB, DH), lambda c, r: (c, r, 0)),
        out_shape=jax.ShapeDtypeStruct((NC, N_NODES, DH), jnp.float32),
    )(H, W)


def _sc_body(hw_hbm, pk_hbm, brep_hbm, out_hbm,
             acc, pk0, pk1, pk2, pk3, rb0, rb1, rb2, semg, sems, semp):
    cid = lax.axis_index("c")
    sid = lax.axis_index("s")

    row0 = sid * ROWS_PER_TILE
    for i in range(5):
        sz = 128 if i < 4 else ROWS_PER_TILE - 4 * 128
        pltpu.sync_copy(brep_hbm.at[cid, pl.ds(0, sz)],
                        acc.at[pl.ds(row0 + i * 128, sz)])
    plsc.subcore_barrier()

    hw_half = hw_hbm.at[cid]
    cbase = sid * CHUNKS_PER_TILE
    pks = [pk0, pk1, pk2, pk3]
    rbs = [rb0, rb1, rb2]

    def scale(pk, rb):
        def body(k, carry):
            wi = plsc.load_gather(pk.at[2], [jnp.full((L,), k, jnp.int32)])
            w = plsc.bitcast(wi, jnp.float32)
            for j in range(DH // L):
                sl = pl.ds(j * L, L)
                rb[k, sl] = rb[k, sl] * w
            return carry
        lax.fori_loop(0, CHUNK, body, 0, unroll=8)

    def wait_scatter(r, p):
        pltpu.make_async_copy(rbs[r], acc.at[pks[p].at[1]], sems).wait()

    def wait_pk(p, c):
        pltpu.make_async_copy(pk_hbm.at[c], pks[p], semp).wait()

    def step(c, r, p, scat_wait):
        if scat_wait:
            wait_scatter((r + 1) % 3, (p + 2) % 4)
        pltpu.async_copy(pk_hbm.at[c + 2], pks[(p + 2) % 4], semp)
        wait_pk((p + 1) % 4, c + 1)
        pltpu.async_copy(hw_half.at[pks[(p + 1) % 4].at[0]],
                         rbs[(r + 1) % 3], semg)
        pltpu.make_async_copy(hw_half.at[pks[p].at[0]], rbs[r], semg).wait()
        scale(pks[p], rbs[r])
        pltpu.async_copy(rbs[r], acc.at[pks[p].at[1]], sems, add=True)

    pltpu.sync_copy(pk_hbm.at[cbase], pk0)
    pltpu.async_copy(hw_half.at[pk0.at[0]], rb0, semg)
    pltpu.async_copy(pk_hbm.at[cbase + 1], pk1, semp)
    step(cbase + 0, 0, 0, False)
    step(cbase + 1, 1, 1, False)

    def body(t, carry):
        c0 = cbase + 12 * t + 2
        for i in range(12):
            step(c0 + i, (2 + i) % 3, (2 + i) % 4, True)
        return carry

    lax.fori_loop(0, (CHUNKS_PER_TILE - 12) // 12, body, 0)
    for i in range(10):
        c = CHUNKS_PER_TILE - 10 + i
        step(cbase + c, c % 3, c % 4, True)

    wait_scatter((CHUNKS_PER_TILE - 2) % 3, (CHUNKS_PER_TILE - 2) % 4)
    wait_scatter((CHUNKS_PER_TILE - 1) % 3, (CHUNKS_PER_TILE - 1) % 4)
    pltpu.make_async_copy(hw_half.at[pk0.at[0]], rb0, semg).wait()
    wait_pk(0, 0)

    plsc.subcore_barrier()

    for i in range(5):
        sz = 128 if i < 4 else ROWS_PER_TILE - 4 * 128
        r = row0 + i * 128
        pltpu.sync_copy(acc.at[pl.ds(r, sz)],
                        out_hbm.at[pl.ds(r, sz), pl.ds(cid * DH, DH)])


def _sc_call(hw2, packed, brep):
    mesh = plsc.VectorSubcoreMesh(core_axis_name="c", subcore_axis_name="s")
    return pl.kernel(
        _sc_body,
        out_type=jax.ShapeDtypeStruct((N_NODES, D_OUT), jnp.float32),
        mesh=mesh,
        compiler_params=pltpu.CompilerParams(use_tc_tiling_on_sc=False,
                                             needs_layout_passes=False),
        scratch_types=[
            pltpu.VMEM_SHARED((N_NODES, DH), jnp.float32),
            pltpu.VMEM((3, CHUNK), jnp.int32),
            pltpu.VMEM((3, CHUNK), jnp.int32),
            pltpu.VMEM((3, CHUNK), jnp.int32),
            pltpu.VMEM((3, CHUNK), jnp.int32),
            pltpu.VMEM((CHUNK, DH), jnp.float32),
            pltpu.VMEM((CHUNK, DH), jnp.float32),
            pltpu.VMEM((CHUNK, DH), jnp.float32),
            pltpu.SemaphoreType.DMA,
            pltpu.SemaphoreType.DMA,
            pltpu.SemaphoreType.DMA,
        ],
    )(hw2, packed, brep)


def kernel(H, edge_index, edge_weight, W, b):
    ei = edge_index.astype(jnp.int32)
    npad = EDGES_PAD - N_EDGES
    spread = jnp.arange(npad, dtype=jnp.int32) % N_NODES
    row = jnp.concatenate([ei[0], spread]).reshape(N_CHUNKS, CHUNK)
    col = jnp.concatenate([ei[1], spread]).reshape(N_CHUNKS, CHUNK)
    ewi = lax.bitcast_convert_type(
        jnp.concatenate([edge_weight, jnp.zeros((npad,), jnp.float32)]),
        jnp.int32).reshape(N_CHUNKS, CHUNK)
    packed = jnp.stack([col, row, ewi], axis=1)
    packed = jnp.concatenate(
        [packed, jnp.zeros((2, 3, CHUNK), jnp.int32)], axis=0)
    hw2 = _matmul_halves(H, W)
    brep = jnp.broadcast_to(b.reshape(NC, 1, DH), (NC, 128, DH))
    return _sc_call(hw2, packed, brep)


# speedup vs baseline: 1.0418x; 1.0418x over previous
import jax
import jax.numpy as jnp
from jax import lax
from jax.experimental import pallas as pl
from jax.experimental.pallas import tpu as pltpu
from jax.experimental.pallas import tpu_sc as plsc

N_NODES = 10000
N_EDGES = 160000
D_IN = 256
D_OUT = 256

NC = 2
NS = 16
L = 16

DH = D_OUT // 2
ROWS_PER_TILE = N_NODES // NS
CHUNK = 120
CHUNKS_PER_TILE = 84
EDGES_PAD = NS * CHUNKS_PER_TILE * CHUNK
N_CHUNKS = EDGES_PAD // CHUNK


def _mm_body(h_ref, w_ref, o_ref):
    o_ref[0] = jnp.dot(h_ref[...], w_ref[...],
                       preferred_element_type=jnp.float32)


def _matmul_halves(H, W):
    RB = 400
    grid = (NC, N_NODES // RB)
    return pl.pallas_call(
        _mm_body,
        grid=grid,
        in_specs=[
            pl.BlockSpec((RB, D_IN), lambda c, r: (r, 0)),
            pl.BlockSpec((D_IN, DH), lambda c, r: (0, c)),
        ],
        out_specs=pl.BlockSpec((1, RB, DH), lambda c, r: (c, r, 0)),
        out_shape=jax.ShapeDtypeStruct((NC, N_NODES, DH), jnp.float32),
    )(H, W)


def _sc_body(hw_hbm, pk_hbm, brep_hbm, out_hbm,
             acc, pk0, pk1, pk2, pk3, rb0, rb1, rb2, semg, sems, semp):
    cid = lax.axis_index("c")
    sid = lax.axis_index("s")

    row0 = sid * ROWS_PER_TILE
    for i in range(5):
        sz = 128 if i < 4 else ROWS_PER_TILE - 4 * 128
        pltpu.sync_copy(brep_hbm.at[cid, pl.ds(0, sz)],
                        acc.at[pl.ds(row0 + i * 128, sz)])
    plsc.subcore_barrier()

    hw_half = hw_hbm.at[cid]
    cbase = sid * CHUNKS_PER_TILE
    pks = [pk0, pk1, pk2, pk3]
    rbs = [rb0, rb1, rb2]

    def scale(pk, rb):
        def body(k, carry):
            wi = plsc.load_gather(pk.at[2], [jnp.full((L,), k, jnp.int32)])
            w = plsc.bitcast(wi, jnp.float32)
            for j in range(DH // L):
                sl = pl.ds(j * L, L)
                rb[k, sl] = rb[k, sl] * w
            return carry
        lax.fori_loop(0, CHUNK, body, 0, unroll=4)

    def wait_scatter(r, p):
        pltpu.make_async_copy(rbs[r], acc.at[pks[p].at[1]], sems).wait()

    def wait_pk(p, c):
        pltpu.make_async_copy(pk_hbm.at[c], pks[p], semp).wait()

    def step(c, r, p, scat_wait):
        if scat_wait:
            wait_scatter((r + 1) % 3, (p + 2) % 4)
        pltpu.async_copy(pk_hbm.at[c + 2], pks[(p + 2) % 4], semp)
        wait_pk((p + 1) % 4, c + 1)
        pltpu.async_copy(hw_half.at[pks[(p + 1) % 4].at[0]],
                         rbs[(r + 1) % 3], semg)
        pltpu.make_async_copy(hw_half.at[pks[p].at[0]], rbs[r], semg).wait()
        scale(pks[p], rbs[r])
        pltpu.async_copy(rbs[r], acc.at[pks[p].at[1]], sems, add=True)

    pltpu.sync_copy(pk_hbm.at[cbase], pk0)
    pltpu.async_copy(hw_half.at[pk0.at[0]], rb0, semg)
    pltpu.async_copy(pk_hbm.at[cbase + 1], pk1, semp)
    step(cbase + 0, 0, 0, False)
    step(cbase + 1, 1, 1, False)

    def body(t, carry):
        c0 = cbase + 12 * t + 2
        for i in range(12):
            step(c0 + i, (2 + i) % 3, (2 + i) % 4, True)
        return carry

    lax.fori_loop(0, (CHUNKS_PER_TILE - 12) // 12, body, 0)
    for i in range(10):
        c = CHUNKS_PER_TILE - 10 + i
        step(cbase + c, c % 3, c % 4, True)

    wait_scatter((CHUNKS_PER_TILE - 2) % 3, (CHUNKS_PER_TILE - 2) % 4)
    wait_scatter((CHUNKS_PER_TILE - 1) % 3, (CHUNKS_PER_TILE - 1) % 4)
    pltpu.make_async_copy(hw_half.at[pk0.at[0]], rb0, semg).wait()
    wait_pk(0, 0)

    plsc.subcore_barrier()

    for i in range(5):
        sz = 128 if i < 4 else ROWS_PER_TILE - 4 * 128
        r = row0 + i * 128
        pltpu.sync_copy(acc.at[pl.ds(r, sz)],
                        out_hbm.at[pl.ds(r, sz), pl.ds(cid * DH, DH)])


def _sc_call(hw2, packed, brep):
    mesh = plsc.VectorSubcoreMesh(core_axis_name="c", subcore_axis_name="s")
    return pl.kernel(
        _sc_body,
        out_type=jax.ShapeDtypeStruct((N_NODES, D_OUT), jnp.float32),
        mesh=mesh,
        compiler_params=pltpu.CompilerParams(use_tc_tiling_on_sc=False,
                                             needs_layout_passes=False),
        scratch_types=[
            pltpu.VMEM_SHARED((N_NODES, DH), jnp.float32),
            pltpu.VMEM((3, CHUNK), jnp.int32),
            pltpu.VMEM((3, CHUNK), jnp.int32),
            pltpu.VMEM((3, CHUNK), jnp.int32),
            pltpu.VMEM((3, CHUNK), jnp.int32),
            pltpu.VMEM((CHUNK, DH), jnp.float32),
            pltpu.VMEM((CHUNK, DH), jnp.float32),
            pltpu.VMEM((CHUNK, DH), jnp.float32),
            pltpu.SemaphoreType.DMA,
            pltpu.SemaphoreType.DMA,
            pltpu.SemaphoreType.DMA,
        ],
    )(hw2, packed, brep)


def kernel(H, edge_index, edge_weight, W, b):
    ei = edge_index.astype(jnp.int32)
    npad = EDGES_PAD - N_EDGES
    spread = jnp.arange(npad, dtype=jnp.int32) % N_NODES
    row = jnp.concatenate([ei[0], spread]).reshape(N_CHUNKS, CHUNK)
    col = jnp.concatenate([ei[1], spread]).reshape(N_CHUNKS, CHUNK)
    ewi = lax.bitcast_convert_type(
        jnp.concatenate([edge_weight, jnp.zeros((npad,), jnp.float32)]),
        jnp.int32).reshape(N_CHUNKS, CHUNK)
    packed = jnp.stack([col, row, ewi], axis=1)
    packed = jnp.concatenate(
        [packed, jnp.zeros((2, 3, CHUNK), jnp.int32)], axis=0)
    hw2 = _matmul_halves(H, W)
    brep = jnp.broadcast_to(b.reshape(NC, 1, DH), (NC, 128, DH))
    return _sc_call(hw2, packed, brep)
